# baseline (speedup 1.0000x reference)
import functools

import jax
import jax.numpy as jnp
import numpy as np
from jax import lax
from jax.experimental import pallas as pl
from jax.experimental.pallas import tpu as pltpu
from jax.experimental.pallas import tpu_sc as plsc

B = 64
Q = 8192
K = 300
KPAD = 304
PBINS = 36
NLM = 10
NVEC_Q = Q // 16
NVEC_K = KPAD // 16

OCC_BASE = 1

_SC_PARAMS = pltpu.CompilerParams(
    needs_layout_passes=False, use_tc_tiling_on_sc=False
)

_IOTA16 = lambda: lax.iota(jnp.int32, 16)


def _gather_elems(table_hbm, idx_ref, dst_ref, n, sem):
  nfull, rem = divmod(n, 128)
  if nfull:
    @pl.loop(0, nfull)
    def _fire(i):
      pltpu.async_copy(
          table_hbm.at[idx_ref.at[pl.ds(i * 128, 128)]],
          dst_ref.at[pl.ds(i * 128, 128)], sem)
  if rem:
    pltpu.async_copy(
        table_hbm.at[idx_ref.at[pl.ds(nfull * 128, rem)]],
        dst_ref.at[pl.ds(nfull * 128, rem)], sem)
  if nfull:
    @pl.loop(0, nfull)
    def _drain(i):
      pltpu.make_async_copy(
          table_hbm.at[idx_ref.at[pl.ds(i * 128, 128)]],
          dst_ref.at[pl.ds(i * 128, 128)], sem).wait()
  if rem:
    pltpu.make_async_copy(
        table_hbm.at[idx_ref.at[pl.ds(nfull * 128, rem)]],
        dst_ref.at[pl.ds(nfull * 128, rem)], sem).wait()


def _sc_topk_gather(scores, boxes_f, lms_f, polar_f, reg_f, mag_f):
  mesh = plsc.VectorSubcoreMesh(core_axis_name="c", subcore_axis_name="s")

  out_type = (
      jax.ShapeDtypeStruct((B, KPAD), jnp.float32),
      jax.ShapeDtypeStruct((B, KPAD), jnp.int32),
      jax.ShapeDtypeStruct((B, 4 * KPAD), jnp.float32),
      jax.ShapeDtypeStruct((B, NLM * KPAD), jnp.float32),
      jax.ShapeDtypeStruct((B, PBINS * KPAD), jnp.float32),
      jax.ShapeDtypeStruct((B, KPAD), jnp.float32),
      jax.ShapeDtypeStruct((B, KPAD), jnp.float32),
  )

  scratch = [
      pltpu.VMEM((Q,), jnp.float32),
      pltpu.VMEM((2048,), jnp.int32),
      pltpu.VMEM((256,), jnp.int32),
      pltpu.VMEM((Q,), jnp.int32),
      pltpu.VMEM((Q,), jnp.int32),
      pltpu.VMEM((Q,), jnp.int32),
      pltpu.VMEM((Q,), jnp.int32),
      pltpu.VMEM((KPAD,), jnp.float32),
      pltpu.VMEM((KPAD,), jnp.int32),
      pltpu.VMEM((KPAD,), jnp.int32),
      pltpu.VMEM((4 * KPAD,), jnp.int32),
      pltpu.VMEM((NLM * KPAD,), jnp.int32),
      pltpu.VMEM((PBINS * KPAD,), jnp.int32),
      pltpu.VMEM((4 * KPAD,), jnp.float32),
      pltpu.VMEM((NLM * KPAD,), jnp.float32),
      pltpu.VMEM((PBINS * KPAD,), jnp.float32),
      pltpu.VMEM((KPAD,), jnp.float32),
      pltpu.VMEM((KPAD,), jnp.float32),
      pltpu.SemaphoreType.DMA,
  ]

  @functools.partial(
      pl.kernel, mesh=mesh, out_type=out_type, scratch_types=scratch,
      compiler_params=_SC_PARAMS,
  )
  def kern(scores_hbm, boxes_hbm, lms_hbm, polar_hbm, reg_hbm, mag_hbm,
           oscore_hbm, oidx_hbm, obox_hbm, olm_hbm, opol_hbm, oreg_hbm,
           omag_hbm, sc_v, hist_v, offs_v, keya_v, idxa_v, keyb_v, idxb_v,
           oscore_v, oidx_v, fidx_v, bxi_v, lmi_v, pli_v, bxg_v, lmg_v,
           plg_v, rgg_v, mgg_v, sem):
    wid = lax.axis_index("s") * 2 + lax.axis_index("c")

    @pl.loop(0, B // 32)
    def _batch(r):
      b = r * 32 + wid

      pltpu.sync_copy(scores_hbm.at[b], sc_v)

      @pl.loop(0, 128)
      def _hz(i):
        hist_v[pl.ds(i * 16, 16)] = jnp.zeros((16,), jnp.int32)

      @pl.loop(0, NVEC_Q)
      def _hist(i):
        key = plsc.bitcast(sc_v[pl.ds(i * 16, 16)], jnp.int32)
        d = lax.shift_right_logical(key, 21)
        occ, last = plsc.scan_count(d)
        cnt = occ + (1 - OCC_BASE)
        plsc.addupdate_scatter(hist_v, [d], cnt, mask=last)

      def _scan_body(i, carry):
        cnt_ge, above, ncand = carry
        base = (127 - i) * 16
        h = hist_v[pl.ds(base, 16)]
        incl = plsc.cumsum(h)
        tot = jnp.max(incl)
        ge = above + tot - incl + h
        is_ge = ge >= K
        cnt_ge = cnt_ge + jnp.sum(jnp.where(is_ge, 1, 0))
        ncand = jnp.minimum(
            ncand, jnp.min(jnp.where(is_ge, ge, jnp.int32(1 << 30))))
        return cnt_ge, above + tot, ncand
      cnt_ge, _, ncand = lax.fori_loop(
          0, 128, _scan_body,
          (jnp.int32(0), jnp.int32(0), jnp.int32(1 << 30)))
      dstar = cnt_ge - 1

      @pl.loop(0, NVEC_K)
      def _zc(i):
        z = jnp.zeros((16,), jnp.int32)
        s = pl.ds(i * 16, 16)
        keya_v[s] = z
        idxa_v[s] = z
        keyb_v[s] = z
        idxb_v[s] = z

      def _compact_body(i, off):
        key = plsc.bitcast(sc_v[pl.ds(i * 16, 16)], jnp.int32)
        d = lax.shift_right_logical(key, 21)
        m = d >= dstar
        mi = jnp.where(m, 1, 0)
        pos = off + plsc.cumsum(mi) - 1
        plsc.store_scatter(keya_v, [pos], key, mask=m)
        plsc.store_scatter(idxa_v, [pos], i * 16 + _IOTA16(), mask=m)
        return off + jnp.sum(mi)
      _ = lax.fori_loop(0, NVEC_Q, _compact_body, jnp.int32(0))

      nvec_c = lax.shift_right_logical(ncand + 15, 4)

      def _radix_pass(shift, src_key, src_idx, dst_key, dst_idx):
        @pl.loop(0, 16)
        def _oz(i):
          offs_v[pl.ds(i * 16, 16)] = jnp.zeros((16,), jnp.int32)

        def _h_body(i, c):
          valid = (i * 16 + _IOTA16()) < ncand
          nk = ~src_key[pl.ds(i * 16, 16)]
          d = lax.shift_right_logical(nk, shift) & 255
          occ, last = plsc.scan_count(d, mask=valid)
          cnt = occ + (1 - OCC_BASE)
          plsc.addupdate_scatter(offs_v, [d], cnt, mask=last & valid)
          return c
        _ = lax.fori_loop(0, nvec_c, _h_body, jnp.int32(0))

        def _p_body(i, c):
          h = offs_v[pl.ds(i * 16, 16)]
          incl = plsc.cumsum(h)
          offs_v[pl.ds(i * 16, 16)] = incl - h + c
          return c + jnp.max(incl)
        _ = lax.fori_loop(0, 16, _p_body, jnp.int32(0))

        def _s_body(i, c):
          valid = (i * 16 + _IOTA16()) < ncand
          k16 = src_key[pl.ds(i * 16, 16)]
          v16 = src_idx[pl.ds(i * 16, 16)]
          d = lax.shift_right_logical(~k16, shift) & 255
          occ, last = plsc.scan_count(d, mask=valid)
          base = plsc.load_gather(offs_v, [d])
          pos = base + occ - OCC_BASE
          plsc.store_scatter(dst_key, [pos], k16, mask=valid)
          plsc.store_scatter(dst_idx, [pos], v16, mask=valid)
          cnt = occ + (1 - OCC_BASE)
          plsc.addupdate_scatter(offs_v, [d], cnt, mask=last & valid)
          return c
        _ = lax.fori_loop(0, nvec_c, _s_body, jnp.int32(0))

      _radix_pass(0, keya_v, idxa_v, keyb_v, idxb_v)
      _radix_pass(8, keyb_v, idxb_v, keya_v, idxa_v)
      _radix_pass(16, keya_v, idxa_v, keyb_v, idxb_v)
      _radix_pass(24, keyb_v, idxb_v, keya_v, idxa_v)

      @pl.loop(0, NVEC_K)
      def _emit(i):
        s = pl.ds(i * 16, 16)
        oscore_v[s] = plsc.bitcast(keya_v[s], jnp.float32)
        oidx_v[s] = idxa_v[s]
        fidx_v[s] = b * Q + idxa_v[s]

      @pl.loop(0, 4 * NVEC_K)
      def _bxidx(i):
        j = i * 16 + _IOTA16()
        src = plsc.load_gather(idxa_v, [lax.shift_right_logical(j, 2)])
        bxi_v[pl.ds(i * 16, 16)] = (b * Q + src) * 4 + (j & 3)

      @pl.loop(0, NLM * NVEC_K)
      def _lmidx(i):
        j = i * 16 + _IOTA16()
        q10 = j // NLM
        src = plsc.load_gather(idxa_v, [q10])
        lmi_v[pl.ds(i * 16, 16)] = (b * Q + src) * NLM + (j - q10 * NLM)

      @pl.loop(0, PBINS * NVEC_K)
      def _plidx(i):
        j = i * 16 + _IOTA16()
        q36 = j // PBINS
        src = plsc.load_gather(idxa_v, [q36])
        pli_v[pl.ds(i * 16, 16)] = (b * Q + src) * PBINS + (j - q36 * PBINS)

      _gather_elems(boxes_hbm, bxi_v, bxg_v, 4 * KPAD, sem)
      _gather_elems(lms_hbm, lmi_v, lmg_v, NLM * KPAD, sem)
      _gather_elems(polar_hbm, pli_v, plg_v, PBINS * KPAD, sem)
      _gather_elems(reg_hbm, fidx_v, rgg_v, KPAD, sem)
      _gather_elems(mag_hbm, fidx_v, mgg_v, KPAD, sem)

      pltpu.sync_copy(oscore_v, oscore_hbm.at[b])
      pltpu.sync_copy(oidx_v, oidx_hbm.at[b])
      pltpu.sync_copy(bxg_v, obox_hbm.at[b])
      pltpu.sync_copy(lmg_v, olm_hbm.at[b])
      pltpu.sync_copy(plg_v, opol_hbm.at[b])
      pltpu.sync_copy(rgg_v, oreg_hbm.at[b])
      pltpu.sync_copy(mgg_v, omag_hbm.at[b])

  return kern(scores, boxes_f, lms_f, polar_f, reg_f, mag_f)


def _tc_post(boxes_g, lms_g, polar_g, reg_g, scale4, scale10):
  bin_size = 2.0 * np.pi / PBINS

  def body(bx_ref, lm_ref, pol_ref, rg_ref, s4_ref, s10_ref,
           obox_ref, olm_ref, obin_ref, oconf_ref, orad_ref, odeg_ref):
    bx = bx_ref[0]
    cx = bx[:, 0:1]
    cy = bx[:, 1:2]
    w = bx[:, 2:3]
    h = bx[:, 3:4]
    xyxy = jnp.concatenate(
        [cx - 0.5 * w, cy - 0.5 * h, cx + 0.5 * w, cy + 0.5 * h], axis=-1)
    obox_ref[0] = xyxy * s4_ref[0]

    olm_ref[0] = lm_ref[0] * s10_ref[0]

    pal = pol_ref[0]
    m = jnp.max(pal, axis=-1, keepdims=True)
    e = jnp.exp(pal - m)
    probs = e / jnp.sum(e, axis=-1, keepdims=True)
    pmax = jnp.max(probs, axis=-1, keepdims=True)
    iota = lax.broadcasted_iota(jnp.int32, probs.shape, 1)
    first = jnp.min(jnp.where(probs == pmax, iota, PBINS), axis=-1)
    obin_ref[0, 0] = first
    oconf_ref[0, 0] = pmax[:, 0]

    centers = (first.astype(jnp.float32) + 0.5) * bin_size
    rad = centers + rg_ref[0, 0] * bin_size
    orad_ref[0, 0] = rad
    odeg_ref[0, 0] = rad * 180.0 / np.pi

  bs = lambda *shape: pl.BlockSpec(
      (1,) + shape, lambda i: (i,) + (0,) * len(shape))
  out_shape = (
      jax.ShapeDtypeStruct((B, KPAD, 4), jnp.float32),
      jax.ShapeDtypeStruct((B, KPAD, NLM), jnp.float32),
      jax.ShapeDtypeStruct((B, 1, KPAD), jnp.int32),
      jax.ShapeDtypeStruct((B, 1, KPAD), jnp.float32),
      jax.ShapeDtypeStruct((B, 1, KPAD), jnp.float32),
      jax.ShapeDtypeStruct((B, 1, KPAD), jnp.float32),
  )
  return pl.pallas_call(
      body,
      grid=(B,),
      in_specs=[bs(KPAD, 4), bs(KPAD, NLM), bs(KPAD, PBINS), bs(1, KPAD),
                bs(1, 4), bs(1, NLM)],
      out_specs=(bs(KPAD, 4), bs(KPAD, NLM), bs(1, KPAD), bs(1, KPAD),
                 bs(1, KPAD), bs(1, KPAD)),
      out_shape=out_shape,
  )(boxes_g, lms_g, polar_g, reg_g.reshape(B, 1, KPAD),
    scale4.reshape(B, 1, 4), scale10.reshape(B, 1, NLM))


def kernel(pred_logits, pred_boxes, pred_landmarks, polar_angle_logits,
           polar_angle_reg, polar_magnitude, orig_target_sizes):
  sizes = orig_target_sizes.astype(jnp.float32)

  scores = jax.nn.sigmoid(pred_logits[..., 0])

  boxes_f = pred_boxes.reshape(-1)
  lms_f = pred_landmarks.reshape(-1)
  polar_f = polar_angle_logits.reshape(-1)
  reg_f = polar_angle_reg.reshape(-1)
  mag_f = polar_magnitude.reshape(-1)

  (tscore, tidx, boxes_g, lms_g, polar_g, reg_g, mag_g) = _sc_topk_gather(
      scores, boxes_f, lms_f, polar_f, reg_f, mag_f)
  del tidx

  boxes_g = boxes_g.reshape(B, KPAD, 4)
  lms_g = lms_g.reshape(B, KPAD, NLM)
  polar_g = polar_g.reshape(B, KPAD, PBINS)

  scale4 = jnp.tile(sizes, (1, 2))
  scale10 = jnp.tile(sizes, (1, NLM // 2))

  boxes_o, lms_o, bin_o, conf_o, rad_o, deg_o = _tc_post(
      boxes_g, lms_g, polar_g, reg_g, scale4, scale10)

  labels = jnp.zeros((B, K), jnp.int32)
  return (labels, boxes_o[:, :K, :], tscore[:, :K], lms_o[:, :K, :],
          bin_o[:, 0, :K], conf_o[:, 0, :K], reg_g[:, :K], mag_g[:, :K],
          rad_o[:, 0, :K], deg_o[:, 0, :K])
